# baseline (speedup 1.0000x reference)
import functools

import jax
import jax.numpy as jnp
from jax import lax
from jax.experimental import pallas as pl
from jax.experimental.pallas import tpu as pltpu
from jax.experimental.pallas import tpu_sc as plsc

N = 10000
C = 256
E = 160000
K = 256
NPAD = 10240
NY = 10112
EPAD = 163840
ROWS_BLK = 128

NC = 2
NS = 16


@functools.cache
def _mesh():
    return plsc.VectorSubcoreMesh(core_axis_name="c", subcore_axis_name="s")


def _topk_body(s_ref, vals_ref, perm_ref):
    NB = NPAD // 1024
    BIG = jnp.int32(1 << 30)
    i8 = lax.broadcasted_iota(jnp.int32, (8, 128), 0)
    i128 = lax.broadcasted_iota(jnp.int32, (8, 128), 1)
    linb = [(i8 + 8 * k) * 128 + i128 for k in range(NB)]
    sb = [s_ref[8 * k:8 * (k + 1), :] for k in range(NB)]
    i256 = lax.broadcasted_iota(jnp.int32, (K,), 0)

    def merge(blocks):
        m, idx = blocks[0], linb[0]
        for k in range(1, NB):
            upd = blocks[k] > m
            m = jnp.where(upd, blocks[k], m)
            idx = jnp.where(upd, linb[k], idx)
        return m, idx

    def body(j, carry):
        blocks, vals, perm = carry
        m_pl, i_pl = merge(blocks)
        m = jnp.max(m_pl)
        idx = jnp.min(jnp.where(m_pl == m, i_pl, BIG))
        vals = jnp.where(i256 == j, m, vals)
        perm = jnp.where(i256 == j, idx, perm)
        blocks = tuple(
            jnp.where(linb[k] == idx, -2.0, blocks[k]) for k in range(NB))
        return blocks, vals, perm

    _, vals, perm = lax.fori_loop(
        0, K, body,
        (tuple(sb), jnp.zeros((K,), jnp.float32), jnp.zeros((K,), jnp.int32)))
    vals_ref[...] = vals[None, :]
    perm_ref[...] = perm[None, :]


def _topk(spad):
    return pl.pallas_call(
        _topk_body,
        out_shape=[jax.ShapeDtypeStruct((1, K), jnp.float32),
                   jax.ShapeDtypeStruct((1, K), jnp.int32)],
    )(spad)


@functools.cache
def _gather_rows_kernel():
    @functools.partial(
        pl.kernel, mesh=_mesh(),
        out_type=jax.ShapeDtypeStruct((K, C), jnp.float32),
        scratch_types=[pltpu.VMEM((8,), jnp.int32),
                       pltpu.VMEM((8, C), jnp.float32),
                       pltpu.SemaphoreType.DMA])
    def _gather_rows(x_hbm, perm_hbm, out_hbm, idx_v, rows_v, sem):
        wid = lax.axis_index("s") * NC + lax.axis_index("c")
        base = wid * 8
        pltpu.sync_copy(perm_hbm.at[pl.ds(base, 8)], idx_v)
        pltpu.async_copy(x_hbm.at[idx_v], rows_v, sem).wait()
        pltpu.sync_copy(rows_v, out_hbm.at[pl.ds(base, 8)])
    return _gather_rows


def _gru_body(xt_ref, ts_ref, wih_ref, whh_ref, bih_ref, bhh_ref, h_ref,
              out_ref):
    xt = xt_ref[...] * ts_ref[...]
    gi = jnp.dot(xt, wih_ref[...], preferred_element_type=jnp.float32)
    gi = gi + bih_ref[...]
    gh = jnp.dot(h_ref[...], whh_ref[...], preferred_element_type=jnp.float32)
    gh = gh + bhh_ref[...]
    r = 1.0 / (1.0 + jnp.exp(-(gi[:, :C] + gh[:, :C])))
    z = 1.0 / (1.0 + jnp.exp(-(gi[:, C:2 * C] + gh[:, C:2 * C])))
    nc = jnp.tanh(gi[:, 2 * C:] + r * gh[:, 2 * C:])
    out_ref[...] = (1.0 - z) * nc + z * h_ref[...]


def _gru(xt_raw, ts, wih_t, whh_t, bih2, bhh2, h):
    return pl.pallas_call(
        _gru_body,
        out_shape=jax.ShapeDtypeStruct((C, C), jnp.float32),
    )(xt_raw, ts, wih_t, whh_t, bih2, bhh2, h)


@functools.cache
def _hist_kernel():
    @functools.partial(
        pl.kernel, mesh=_mesh(),
        out_type=jax.ShapeDtypeStruct((NC, NPAD), jnp.float32),
        scratch_types=[pltpu.VMEM((EPAD // NC // NS // 128, 128), jnp.int32),
                       pltpu.VMEM((128,), jnp.float32),
                       pltpu.VMEM((NPAD // NS,), jnp.float32),
                       pltpu.VMEM_SHARED((NPAD,), jnp.float32)])
    def _hist_sc(col2_hbm, z_hbm, ones_hbm, out_hbm, colbuf, onesv, hbuf,
                 hist):
        cid = lax.axis_index("c")
        tid = lax.axis_index("s")
        chunk = NPAD // NS
        pltpu.sync_copy(z_hbm.at[pl.ds(tid * chunk, chunk)], hbuf)
        pltpu.sync_copy(hbuf, hist.at[pl.ds(tid * chunk, chunk)])
        pltpu.sync_copy(ones_hbm, onesv)
        nb = EPAD // NC // NS // 128
        pltpu.sync_copy(col2_hbm.at[pl.ds(cid * NS * nb + tid * nb, nb)],
                        colbuf)
        plsc.subcore_barrier()

        def body(i, _):
            pltpu.sync_copy(onesv, hist.at[colbuf.at[i]], add=True)
            return 0

        lax.fori_loop(0, nb, body, 0)
        plsc.subcore_barrier()
        pltpu.sync_copy(hist.at[pl.ds(tid * chunk, chunk)], hbuf)
        pltpu.sync_copy(hbuf, out_hbm.at[cid, pl.ds(tid * chunk, chunk)])
    return _hist_sc


@functools.cache
def _scatter_kernel():
    nb = EPAD // NS // 128
    hb = nb // 2
    @functools.partial(
        pl.kernel, mesh=_mesh(),
        out_type=jax.ShapeDtypeStruct((NC, NPAD, 128), jnp.float32),
        scratch_types=[pltpu.VMEM((hb, 128), jnp.int32),
                       pltpu.VMEM((hb, 128), jnp.int32),
                       pltpu.VMEM((128, 128), jnp.float32),
                       pltpu.VMEM((128, 128), jnp.float32),
                       pltpu.VMEM_SHARED((NPAD, 128), jnp.float32),
                       pltpu.SemaphoreType.DMA,
                       pltpu.SemaphoreType.DMA])
    def _scatter_sc(y_hbm, rows_hbm, col_hbm, z_hbm, out_hbm,
                    rowbuf, colbuf, gbuf0, gbuf1, acc, sem0, sem1):
        cid = lax.axis_index("c")
        tid = lax.axis_index("s")
        rows_per_tile = NPAD // NS
        r0 = tid * rows_per_tile
        pltpu.sync_copy(z_hbm, gbuf0)
        for j in range(rows_per_tile // 128):
            pltpu.sync_copy(gbuf0, acc.at[pl.ds(r0 + j * 128, 128)])
        plsc.subcore_barrier()

        def gather(j, buf, sem):
            return pltpu.async_copy(y_hbm.at[rowbuf.at[j]], buf, sem)

        def scat(j, buf):
            pltpu.sync_copy(buf, acc.at[colbuf.at[j]], add=True)

        for h in range(2):
            base = tid * nb + h * hb
            pltpu.sync_copy(rows_hbm.at[cid, pl.ds(base, hb)], rowbuf)
            pltpu.sync_copy(col_hbm.at[pl.ds(base, hb)], colbuf)
            gather(0, gbuf0, sem0)

            def body(i, _):
                a = 2 * i
                gather(a + 1, gbuf1, sem1)
                pltpu.make_async_copy(
                    y_hbm.at[rowbuf.at[a]], gbuf0, sem0).wait()
                scat(a, gbuf0)
                gather(a + 2, gbuf0, sem0)
                pltpu.make_async_copy(
                    y_hbm.at[rowbuf.at[a + 1]], gbuf1, sem1).wait()
                scat(a + 1, gbuf1)
                return 0

            lax.fori_loop(0, hb // 2 - 1, body, 0)
            gather(hb - 1, gbuf1, sem1)
            pltpu.make_async_copy(
                y_hbm.at[rowbuf.at[hb - 2]], gbuf0, sem0).wait()
            scat(hb - 2, gbuf0)
            pltpu.make_async_copy(
                y_hbm.at[rowbuf.at[hb - 1]], gbuf1, sem1).wait()
            scat(hb - 1, gbuf1)

        plsc.subcore_barrier()
        for j in range(rows_per_tile // 128):
            pltpu.sync_copy(acc.at[pl.ds(r0 + j * 128, 128)], gbuf0)
            pltpu.sync_copy(gbuf0, out_hbm.at[cid, pl.ds(r0 + j * 128, 128)])
    return _scatter_sc


def _scale_mm_body(x_ref, w_ref, h_ref, y_ref, os_ref):
    deg = h_ref[:, 0:1] + h_ref[:, 1:2] + 1.0
    dinv = lax.rsqrt(deg)
    xs = x_ref[...] * dinv
    xw = jnp.dot(xs, w_ref[...], preferred_element_type=jnp.float32)
    y_ref[0] = xw[:, :128]
    y_ref[1] = xw[:, 128:]
    os_ref[...] = xw * dinv


def _scale_mm(x, w, hist_t):
    grid = (NY // ROWS_BLK,)
    return pl.pallas_call(
        _scale_mm_body,
        grid=grid,
        in_specs=[pl.BlockSpec((ROWS_BLK, C), lambda i: (i, 0)),
                  pl.BlockSpec((C, C), lambda i: (0, 0)),
                  pl.BlockSpec((ROWS_BLK, NC), lambda i: (i, 0))],
        out_specs=[pl.BlockSpec((NC, ROWS_BLK, 128), lambda i: (0, i, 0)),
                   pl.BlockSpec((ROWS_BLK, C), lambda i: (i, 0))],
        out_shape=[jax.ShapeDtypeStruct((NC, NY, 128), jnp.float32),
                   jax.ShapeDtypeStruct((NY, C), jnp.float32)],
    )(x, w, hist_t)


def _combine_body(s_ref, h_ref, os_ref, o_ref):
    deg = h_ref[:, 0:1] + h_ref[:, 1:2] + 1.0
    dinv = lax.rsqrt(deg)
    agg = jnp.concatenate([s_ref[0], s_ref[1]], axis=1)
    o_ref[...] = agg * dinv + os_ref[...]


def _combine(s, hist_t, oself):
    grid = (pl.cdiv(N, ROWS_BLK),)
    return pl.pallas_call(
        _combine_body,
        grid=grid,
        in_specs=[pl.BlockSpec((NC, ROWS_BLK, 128), lambda i: (0, i, 0)),
                  pl.BlockSpec((ROWS_BLK, NC), lambda i: (i, 0)),
                  pl.BlockSpec((ROWS_BLK, C), lambda i: (i, 0))],
        out_specs=pl.BlockSpec((ROWS_BLK, C), lambda i: (i, 0)),
        out_shape=jax.ShapeDtypeStruct((N, C), jnp.float32),
    )(s, hist_t, oself)


def kernel(X, edge_index, p, W_ih, W_hh, b_ih, b_hh, initial_weight):
    row = edge_index[0].astype(jnp.int32)
    col = edge_index[1].astype(jnp.int32)

    pad_e = EPAD - E
    pidx = lax.iota(jnp.int32, pad_e)
    row_pad = jnp.concatenate([row, pidx % N])
    col_pad = jnp.concatenate([col, N + pidx % (NPAD - N)])
    rows2 = jnp.stack([row_pad, row_pad + NY]).reshape(NC, EPAD // 128, 128)
    col2 = col_pad.reshape(EPAD // 128, 128)

    zeros2d = jnp.zeros((128, 128), jnp.float32)
    zeros1d = jnp.zeros((NPAD,), jnp.float32)
    ones1 = jnp.ones((128,), jnp.float32)

    score = jnp.tanh((X @ p) / (jnp.linalg.norm(p) + 1e-16))
    spad = jnp.concatenate(
        [score, jnp.full((NPAD - N,), -2.0, jnp.float32)]).reshape(
            NPAD // 128, 128)

    vals2, perm2 = _topk(spad)
    xt_raw = _gather_rows_kernel()(X, perm2.reshape(K))
    w = _gru(xt_raw, vals2.reshape(K, 1), W_ih.T, W_hh.T,
             b_ih.reshape(1, 3 * C), b_hh.reshape(1, 3 * C), initial_weight)

    hist = _hist_kernel()(col2, zeros1d, ones1)
    hist_t = hist.T

    y, oself = _scale_mm(X, w, hist_t)
    s = _scatter_kernel()(y.reshape(NC * NY, 128), rows2, col2, zeros2d)
    return _combine(s, hist_t, oself)


# speedup vs baseline: 1.0105x; 1.0105x over previous
import functools

import jax
import jax.numpy as jnp
from jax import lax
from jax.experimental import pallas as pl
from jax.experimental.pallas import tpu as pltpu
from jax.experimental.pallas import tpu_sc as plsc

N = 10000
C = 256
E = 160000
K = 256
NPAD = 10240
NY = 10112
EPAD = 163840
ROWS_BLK = 128

NC = 2
NS = 16


@functools.cache
def _mesh():
    return plsc.VectorSubcoreMesh(core_axis_name="c", subcore_axis_name="s")


def _topk_body(s_ref, vals_ref, perm_ref):
    NB = NPAD // 1024
    BIG = jnp.int32(1 << 30)
    i8 = lax.broadcasted_iota(jnp.int32, (8, 128), 0)
    i128 = lax.broadcasted_iota(jnp.int32, (8, 128), 1)
    linb = [(i8 + 8 * k) * 128 + i128 for k in range(NB)]
    sb = [s_ref[8 * k:8 * (k + 1), :] for k in range(NB)]
    i256 = lax.broadcasted_iota(jnp.int32, (K,), 0)

    def merge(blocks):
        m, idx = blocks[0], linb[0]
        for k in range(1, NB):
            upd = blocks[k] > m
            m = jnp.where(upd, blocks[k], m)
            idx = jnp.where(upd, linb[k], idx)
        return m, idx

    def body(j, carry):
        blocks, vals, perm = carry
        m_pl, i_pl = merge(blocks)
        m = jnp.max(m_pl)
        idx = jnp.min(jnp.where(m_pl == m, i_pl, BIG))
        vals = jnp.where(i256 == j, m, vals)
        perm = jnp.where(i256 == j, idx, perm)
        blocks = tuple(
            jnp.where(linb[k] == idx, -2.0, blocks[k]) for k in range(NB))
        return blocks, vals, perm

    _, vals, perm = lax.fori_loop(
        0, K, body,
        (tuple(sb), jnp.zeros((K,), jnp.float32), jnp.zeros((K,), jnp.int32)))
    vals_ref[...] = vals[None, :]
    perm_ref[...] = perm[None, :]


def _topk(spad):
    return pl.pallas_call(
        _topk_body,
        out_shape=[jax.ShapeDtypeStruct((1, K), jnp.float32),
                   jax.ShapeDtypeStruct((1, K), jnp.int32)],
    )(spad)


@functools.cache
def _gather_rows_kernel():
    @functools.partial(
        pl.kernel, mesh=_mesh(),
        out_type=jax.ShapeDtypeStruct((K, C), jnp.float32),
        scratch_types=[pltpu.VMEM((8,), jnp.int32),
                       pltpu.VMEM((8, C), jnp.float32),
                       pltpu.SemaphoreType.DMA])
    def _gather_rows(x_hbm, perm_hbm, out_hbm, idx_v, rows_v, sem):
        wid = lax.axis_index("s") * NC + lax.axis_index("c")
        base = wid * 8
        pltpu.sync_copy(perm_hbm.at[pl.ds(base, 8)], idx_v)
        pltpu.async_copy(x_hbm.at[idx_v], rows_v, sem).wait()
        pltpu.sync_copy(rows_v, out_hbm.at[pl.ds(base, 8)])
    return _gather_rows


@functools.cache
def _hist_kernel():
    @functools.partial(
        pl.kernel, mesh=_mesh(),
        out_type=jax.ShapeDtypeStruct((NC, NPAD), jnp.float32),
        scratch_types=[pltpu.VMEM((EPAD // NC // NS // 128, 128), jnp.int32),
                       pltpu.VMEM((128,), jnp.float32),
                       pltpu.VMEM((NPAD // NS,), jnp.float32),
                       pltpu.VMEM_SHARED((NPAD,), jnp.float32)])
    def _hist_sc(col2_hbm, z_hbm, ones_hbm, out_hbm, colbuf, onesv, hbuf,
                 hist):
        cid = lax.axis_index("c")
        tid = lax.axis_index("s")
        chunk = NPAD // NS
        pltpu.sync_copy(z_hbm.at[pl.ds(tid * chunk, chunk)], hbuf)
        pltpu.sync_copy(hbuf, hist.at[pl.ds(tid * chunk, chunk)])
        pltpu.sync_copy(ones_hbm, onesv)
        nb = EPAD // NC // NS // 128
        pltpu.sync_copy(col2_hbm.at[pl.ds(cid * NS * nb + tid * nb, nb)],
                        colbuf)
        plsc.subcore_barrier()

        def body(i, _):
            pltpu.sync_copy(onesv, hist.at[colbuf.at[i]], add=True)
            return 0

        lax.fori_loop(0, nb, body, 0)
        plsc.subcore_barrier()
        pltpu.sync_copy(hist.at[pl.ds(tid * chunk, chunk)], hbuf)
        pltpu.sync_copy(hbuf, out_hbm.at[cid, pl.ds(tid * chunk, chunk)])
    return _hist_sc


@functools.cache
def _scatter_kernel():
    nb = EPAD // NS // 128
    hb = nb // 2
    @functools.partial(
        pl.kernel, mesh=_mesh(),
        out_type=jax.ShapeDtypeStruct((NC, NPAD, 128), jnp.float32),
        scratch_types=[pltpu.VMEM((hb, 128), jnp.int32),
                       pltpu.VMEM((hb, 128), jnp.int32),
                       pltpu.VMEM((128, 128), jnp.float32),
                       pltpu.VMEM((128, 128), jnp.float32),
                       pltpu.VMEM_SHARED((NPAD, 128), jnp.float32),
                       pltpu.SemaphoreType.DMA,
                       pltpu.SemaphoreType.DMA])
    def _scatter_sc(y_hbm, rows_hbm, col_hbm, z_hbm, out_hbm,
                    rowbuf, colbuf, gbuf0, gbuf1, acc, sem0, sem1):
        cid = lax.axis_index("c")
        tid = lax.axis_index("s")
        rows_per_tile = NPAD // NS
        r0 = tid * rows_per_tile
        pltpu.sync_copy(z_hbm, gbuf0)
        for j in range(rows_per_tile // 128):
            pltpu.sync_copy(gbuf0, acc.at[pl.ds(r0 + j * 128, 128)])
        plsc.subcore_barrier()

        def gather(j, buf, sem):
            return pltpu.async_copy(y_hbm.at[rowbuf.at[j]], buf, sem)

        def scat(j, buf):
            pltpu.sync_copy(buf, acc.at[colbuf.at[j]], add=True)

        for h in range(2):
            base = tid * nb + h * hb
            pltpu.sync_copy(rows_hbm.at[cid, pl.ds(base, hb)], rowbuf)
            pltpu.sync_copy(col_hbm.at[pl.ds(base, hb)], colbuf)
            gather(0, gbuf0, sem0)

            def body(i, _):
                a = 2 * i
                gather(a + 1, gbuf1, sem1)
                pltpu.make_async_copy(
                    y_hbm.at[rowbuf.at[a]], gbuf0, sem0).wait()
                scat(a, gbuf0)
                gather(a + 2, gbuf0, sem0)
                pltpu.make_async_copy(
                    y_hbm.at[rowbuf.at[a + 1]], gbuf1, sem1).wait()
                scat(a + 1, gbuf1)
                return 0

            lax.fori_loop(0, hb // 2 - 1, body, 0)
            gather(hb - 1, gbuf1, sem1)
            pltpu.make_async_copy(
                y_hbm.at[rowbuf.at[hb - 2]], gbuf0, sem0).wait()
            scat(hb - 2, gbuf0)
            pltpu.make_async_copy(
                y_hbm.at[rowbuf.at[hb - 1]], gbuf1, sem1).wait()
            scat(hb - 1, gbuf1)

        plsc.subcore_barrier()
        for j in range(rows_per_tile // 128):
            pltpu.sync_copy(acc.at[pl.ds(r0 + j * 128, 128)], gbuf0)
            pltpu.sync_copy(gbuf0, out_hbm.at[cid, pl.ds(r0 + j * 128, 128)])
    return _scatter_sc


def _scale_mm_body(x_ref, xt_ref, ts_ref, wih_ref, whh_ref, bih_ref,
                   bhh_ref, h_ref, hist_ref, y_ref, os_ref, w_scr):
    @pl.when(pl.program_id(0) == 0)
    def _():
        xt = xt_ref[...] * ts_ref[...]
        dn = (((1,), (1,)), ((), ()))
        gi = lax.dot_general(xt, wih_ref[...], dn,
                             preferred_element_type=jnp.float32)
        gi = gi + bih_ref[...]
        gh = lax.dot_general(h_ref[...], whh_ref[...], dn,
                             preferred_element_type=jnp.float32)
        gh = gh + bhh_ref[...]
        r = 1.0 / (1.0 + jnp.exp(-(gi[:, :C] + gh[:, :C])))
        z = 1.0 / (1.0 + jnp.exp(-(gi[:, C:2 * C] + gh[:, C:2 * C])))
        nc = jnp.tanh(gi[:, 2 * C:] + r * gh[:, 2 * C:])
        w_scr[...] = (1.0 - z) * nc + z * h_ref[...]

    deg = hist_ref[:, 0:1] + hist_ref[:, 1:2] + 1.0
    dinv = lax.rsqrt(deg)
    xs = x_ref[...] * dinv
    xw = jnp.dot(xs, w_scr[...], preferred_element_type=jnp.float32)
    y_ref[0] = xw[:, :128]
    y_ref[1] = xw[:, 128:]
    os_ref[...] = xw * dinv


def _scale_mm(x, xt_raw, ts, wih, whh, bih2, bhh2, h, hist_t):
    grid = (NY // ROWS_BLK,)
    zero = lambda i: (0, 0)
    return pl.pallas_call(
        _scale_mm_body,
        grid=grid,
        in_specs=[pl.BlockSpec((ROWS_BLK, C), lambda i: (i, 0)),
                  pl.BlockSpec((K, C), zero),
                  pl.BlockSpec((K, 1), zero),
                  pl.BlockSpec((3 * C, C), zero),
                  pl.BlockSpec((3 * C, C), zero),
                  pl.BlockSpec((1, 3 * C), zero),
                  pl.BlockSpec((1, 3 * C), zero),
                  pl.BlockSpec((C, C), zero),
                  pl.BlockSpec((ROWS_BLK, NC), lambda i: (i, 0))],
        out_specs=[pl.BlockSpec((NC, ROWS_BLK, 128), lambda i: (0, i, 0)),
                   pl.BlockSpec((ROWS_BLK, C), lambda i: (i, 0))],
        out_shape=[jax.ShapeDtypeStruct((NC, NY, 128), jnp.float32),
                   jax.ShapeDtypeStruct((NY, C), jnp.float32)],
        scratch_shapes=[pltpu.VMEM((C, C), jnp.float32)],
    )(x, xt_raw, ts, wih, whh, bih2, bhh2, h, hist_t)


def _combine_body(s_ref, h_ref, os_ref, o_ref):
    deg = h_ref[:, 0:1] + h_ref[:, 1:2] + 1.0
    dinv = lax.rsqrt(deg)
    agg = jnp.concatenate([s_ref[0], s_ref[1]], axis=1)
    o_ref[...] = agg * dinv + os_ref[...]


def _combine(s, hist_t, oself):
    grid = (pl.cdiv(N, ROWS_BLK),)
    return pl.pallas_call(
        _combine_body,
        grid=grid,
        in_specs=[pl.BlockSpec((NC, ROWS_BLK, 128), lambda i: (0, i, 0)),
                  pl.BlockSpec((ROWS_BLK, NC), lambda i: (i, 0)),
                  pl.BlockSpec((ROWS_BLK, C), lambda i: (i, 0))],
        out_specs=pl.BlockSpec((ROWS_BLK, C), lambda i: (i, 0)),
        out_shape=jax.ShapeDtypeStruct((N, C), jnp.float32),
    )(s, hist_t, oself)


def kernel(X, edge_index, p, W_ih, W_hh, b_ih, b_hh, initial_weight):
    row = edge_index[0].astype(jnp.int32)
    col = edge_index[1].astype(jnp.int32)

    pad_e = EPAD - E
    pidx = lax.iota(jnp.int32, pad_e)
    row_pad = jnp.concatenate([row, pidx % N])
    col_pad = jnp.concatenate([col, N + pidx % (NPAD - N)])
    rows2 = jnp.stack([row_pad, row_pad + NY]).reshape(NC, EPAD // 128, 128)
    col2 = col_pad.reshape(EPAD // 128, 128)

    zeros2d = jnp.zeros((128, 128), jnp.float32)
    zeros1d = jnp.zeros((NPAD,), jnp.float32)
    ones1 = jnp.ones((128,), jnp.float32)

    score = jnp.tanh((X @ p) / (jnp.linalg.norm(p) + 1e-16))
    spad = jnp.concatenate(
        [score, jnp.full((NPAD - N,), -2.0, jnp.float32)]).reshape(
            NPAD // 128, 128)

    vals2, perm2 = _topk(spad)
    xt_raw = _gather_rows_kernel()(X, perm2.reshape(K))

    hist = _hist_kernel()(col2, zeros1d, ones1)
    hist_t = hist.T

    y, oself = _scale_mm(X, xt_raw, vals2.reshape(K, 1), W_ih, W_hh,
                         b_ih.reshape(1, 3 * C), b_hh.reshape(1, 3 * C),
                         initial_weight, hist_t)
    s = _scatter_kernel()(y.reshape(NC * NY, 128), rows2, col2, zeros2d)
    return _combine(s, hist_t, oself)
